# baseline (speedup 1.0000x reference)
import functools

import jax
import jax.numpy as jnp
from jax import lax
from jax.experimental import pallas as pl
from jax.experimental.pallas import tpu as pltpu
from jax.experimental.pallas import tpu_sc as plsc

N_SEG, D = 50000, 512
B, L = 16, 2048
ROWS = B * L
NW = 32
C = 64
PCH = L // C
NCHUNK = B
VPR = D // 16


def _body(table_hbm, ids_hbm, len_hbm, out_hbm,
          idx_v, rows0, rows1, zbuf, len_v, gsem0, gsem1, isem, wsem0, wsem1):
    nc = 2
    wid = lax.axis_index("s") * nc + lax.axis_index("c")

    pltpu.sync_copy(len_hbm, len_v.at[pl.ds(0, 16)])

    starts = []
    for j in range(NCHUNK):
        p = jnp.bitwise_and(wid + 2 * j, PCH - 1)
        starts.append(j * L + p * C)

    for j in range(NCHUNK):
        pltpu.async_copy(
            ids_hbm.at[pl.ds(starts[j], C)], idx_v.at[pl.ds(j * C, C)], isem)
    pltpu.make_async_copy(
        ids_hbm.at[pl.ds(0, NCHUNK * C)], idx_v, isem).wait()

    lv = len_v[pl.ds(0, 16)]
    nvs = []
    for j in range(NCHUNK):
        nvs.append(jnp.clip(lv[j] - (starts[j] - j * L), 0, C))

    rows = (rows0, rows1)
    gsems = (gsem0, gsem1)
    wsems = (wsem0, wsem1)

    def issue_gather(i):
        @pl.when(nvs[i] > 0)
        def _():
            pltpu.async_copy(
                table_hbm.at[idx_v.at[pl.ds(i * C, C)]], rows[i % 2],
                gsems[i % 2])

    issue_gather(0)

    def _zrow(r, _):
        def _zcol(jj, _):
            zbuf[r, pl.ds(jj * 16, 16)] = jnp.zeros((16,), jnp.float32)
            return 0
        return lax.fori_loop(0, VPR, _zcol, 0)
    lax.fori_loop(0, C, _zrow, 0)

    for i in range(1, NCHUNK + 1):
        if i < NCHUNK:
            if i >= 2:
                pltpu.make_async_copy(
                    rows[i % 2], out_hbm.at[pl.ds(0, C)], wsems[i % 2]).wait()
            issue_gather(i)

        j = i - 1
        buf = rows[j % 2]
        nv = nvs[j]

        @pl.when(nv > 0)
        def _gathered():
            pltpu.make_async_copy(
                table_hbm.at[idx_v.at[pl.ds(j * C, C)]], buf,
                gsems[j % 2]).wait()

            @pl.when(nv < C)
            def _tail():
                def _ztail(k, _):
                    r = lax.shift_right_logical(k, 5)
                    col = jnp.bitwise_and(k, VPR - 1) * 16
                    buf[r, pl.ds(col, 16)] = jnp.zeros((16,), jnp.float32)
                    return 0
                lax.fori_loop(nv * VPR, C * VPR, _ztail, 0)

            pltpu.async_copy(buf, out_hbm.at[pl.ds(starts[j], C)], wsems[j % 2])

        @pl.when(nv <= 0)
        def _empty():
            pltpu.async_copy(
                zbuf, out_hbm.at[pl.ds(starts[j], C)], wsems[j % 2])

    pltpu.make_async_copy(rows0, out_hbm.at[pl.ds(0, C)], wsem0).wait()
    pltpu.make_async_copy(rows1, out_hbm.at[pl.ds(0, C)], wsem1).wait()


@jax.jit
def _sembed(table, ids_flat, lengths):
    mesh = plsc.VectorSubcoreMesh(core_axis_name="c", subcore_axis_name="s")
    f = functools.partial(
        pl.kernel,
        out_type=jax.ShapeDtypeStruct((ROWS, D), jnp.float32),
        mesh=mesh,
        scratch_types=[
            pltpu.VMEM((NCHUNK * C,), jnp.int32),
            pltpu.VMEM((C, D), jnp.float32),
            pltpu.VMEM((C, D), jnp.float32),
            pltpu.VMEM((C, D), jnp.float32),
            pltpu.VMEM((32,), jnp.int32),
            pltpu.SemaphoreType.DMA,
            pltpu.SemaphoreType.DMA,
            pltpu.SemaphoreType.DMA,
            pltpu.SemaphoreType.DMA,
            pltpu.SemaphoreType.DMA,
        ],
    )(_body)
    return f(table, ids_flat, lengths)


def kernel(seg_embeddings, edge_index, edge_weights, traj_ids, traj_lengths):
    del edge_index, edge_weights
    out = _sembed(seg_embeddings, traj_ids.reshape(ROWS), traj_lengths)
    return out.reshape(B, L, D)


# speedup vs baseline: 1.0763x; 1.0763x over previous
import functools

import jax
import jax.numpy as jnp
from jax import lax
from jax.experimental import pallas as pl
from jax.experimental.pallas import tpu as pltpu
from jax.experimental.pallas import tpu_sc as plsc

N_SEG, D = 50000, 512
B, L = 16, 2048
ROWS = B * L
NW = 32
C = 64
PCH = L // C
NCHUNK = B
VPR = D // 16


def _body(table_hbm, ids_hbm, len_hbm, out_hbm,
          idx_v, rows0, rows1, rows2, zbuf, len_v,
          gsem0, gsem1, gsem2, isem, wsem0, wsem1, wsem2):
    nc = 2
    wid = lax.axis_index("s") * nc + lax.axis_index("c")

    pltpu.sync_copy(len_hbm, len_v.at[pl.ds(0, 16)])

    starts = []
    for j in range(NCHUNK):
        p = jnp.bitwise_and(wid + 2 * j, PCH - 1)
        starts.append(j * L + p * C)

    for j in range(NCHUNK):
        pltpu.async_copy(
            ids_hbm.at[pl.ds(starts[j], C)], idx_v.at[pl.ds(j * C, C)], isem)
    pltpu.make_async_copy(
        ids_hbm.at[pl.ds(0, NCHUNK * C)], idx_v, isem).wait()

    lv = len_v[pl.ds(0, 16)]
    nvs = []
    for j in range(NCHUNK):
        nvs.append(jnp.clip(lv[j] - (starts[j] - j * L), 0, C))

    rows = (rows0, rows1, rows2)
    gsems = (gsem0, gsem1, gsem2)
    wsems = (wsem0, wsem1, wsem2)
    nbuf = 3

    def issue_gather(i):
        @pl.when(nvs[i] > 0)
        def _():
            pltpu.async_copy(
                table_hbm.at[idx_v.at[pl.ds(i * C, C)]], rows[i % 3],
                gsems[i % 3])

    issue_gather(0)

    def _zrow(r, _):
        def _zcol(jj, _):
            zbuf[r, pl.ds(jj * 16, 16)] = jnp.zeros((16,), jnp.float32)
            return 0
        return lax.fori_loop(0, VPR, _zcol, 0)
    lax.fori_loop(0, C // 2, _zrow, 0)

    for i in range(1, NCHUNK + 1):
        if i < NCHUNK:
            if i >= 3:
                pltpu.make_async_copy(
                    rows[i % 3], out_hbm.at[pl.ds(0, C)], wsems[i % 3]).wait()
            issue_gather(i)

        j = i - 1
        buf = rows[j % 3]
        nv = nvs[j]

        @pl.when(nv > 0)
        def _gathered():
            pltpu.make_async_copy(
                table_hbm.at[idx_v.at[pl.ds(j * C, C)]], buf,
                gsems[j % 3]).wait()

            @pl.when(nv < C)
            def _tail():
                def _ztail(k, _):
                    r = lax.shift_right_logical(k, 5)
                    col = jnp.bitwise_and(k, VPR - 1) * 16
                    buf[r, pl.ds(col, 16)] = jnp.zeros((16,), jnp.float32)
                    return 0
                lax.fori_loop(nv * VPR, C * VPR, _ztail, 0)

            pltpu.async_copy(buf, out_hbm.at[pl.ds(starts[j], C)], wsems[j % 3])

        @pl.when(nv <= 0)
        def _empty():
            pltpu.async_copy(
                zbuf, out_hbm.at[pl.ds(starts[j], C // 2)], wsems[j % 3])
            pltpu.async_copy(
                zbuf, out_hbm.at[pl.ds(starts[j] + C // 2, C // 2)],
                wsems[j % 3])

    pltpu.make_async_copy(rows0, out_hbm.at[pl.ds(0, C)], wsem0).wait()
    pltpu.make_async_copy(rows1, out_hbm.at[pl.ds(0, C)], wsem1).wait()
    pltpu.make_async_copy(rows2, out_hbm.at[pl.ds(0, C)], wsem2).wait()


@jax.jit
def _sembed(table, ids_flat, lengths):
    mesh = plsc.VectorSubcoreMesh(core_axis_name="c", subcore_axis_name="s")
    f = functools.partial(
        pl.kernel,
        out_type=jax.ShapeDtypeStruct((ROWS, D), jnp.float32),
        mesh=mesh,
        scratch_types=[
            pltpu.VMEM((NCHUNK * C,), jnp.int32),
            pltpu.VMEM((C, D), jnp.float32),
            pltpu.VMEM((C, D), jnp.float32),
            pltpu.VMEM((C, D), jnp.float32),
            pltpu.VMEM((C // 2, D), jnp.float32),
            pltpu.VMEM((32,), jnp.int32),
            pltpu.SemaphoreType.DMA,
            pltpu.SemaphoreType.DMA,
            pltpu.SemaphoreType.DMA,
            pltpu.SemaphoreType.DMA,
            pltpu.SemaphoreType.DMA,
            pltpu.SemaphoreType.DMA,
            pltpu.SemaphoreType.DMA,
        ],
    )(_body)
    return f(table, ids_flat, lengths)


def kernel(seg_embeddings, edge_index, edge_weights, traj_ids, traj_lengths):
    del edge_index, edge_weights
    out = _sembed(seg_embeddings, traj_ids.reshape(ROWS), traj_lengths)
    return out.reshape(B, L, D)
